# initial kernel scaffold (unmeasured)
import jax
import jax.numpy as jnp
from jax import lax
from jax.experimental import pallas as pl
from jax.experimental.pallas import tpu as pltpu


def kernel(
    x,
):
    def body(*refs):
        pass

    out_shape = jax.ShapeDtypeStruct(..., jnp.float32)
    return pl.pallas_call(body, out_shape=out_shape)(...)


# baseline (device time: 30016 ns/iter reference)
import jax
import jax.numpy as jnp
from jax import lax
from jax.experimental import pallas as pl
from jax.experimental.pallas import tpu as pltpu


def kernel(x):
    m, n = x.shape

    def body(x_ref, out_ref, send_sem, recv_sem):
        mx = lax.axis_index("x")
        my = lax.axis_index("y")
        mz = lax.axis_index("z")
        xpeer = (1 - mx, my, mz)

        barrier_sem = pltpu.get_barrier_semaphore()
        pl.semaphore_signal(
            barrier_sem, inc=1,
            device_id=xpeer, device_id_type=pl.DeviceIdType.MESH,
        )
        pl.semaphore_wait(barrier_sem, 1)

        out_ref[pl.ds(mx * m, m), :] = x_ref[:, :]

        rdma = pltpu.make_async_remote_copy(
            src_ref=x_ref,
            dst_ref=out_ref.at[pl.ds(mx * m, m), :],
            send_sem=send_sem,
            recv_sem=recv_sem,
            device_id=xpeer,
            device_id_type=pl.DeviceIdType.MESH,
        )
        rdma.start()
        rdma.wait()

    return pl.pallas_call(
        body,
        out_shape=jax.ShapeDtypeStruct((2 * m, n), x.dtype),
        in_specs=[pl.BlockSpec(memory_space=pltpu.VMEM)],
        out_specs=pl.BlockSpec(memory_space=pltpu.VMEM),
        scratch_shapes=[
            pltpu.SemaphoreType.DMA,
            pltpu.SemaphoreType.DMA,
        ],
        compiler_params=pltpu.CompilerParams(collective_id=0),
    )(x)


# device time: 22805 ns/iter; 1.3162x vs baseline; 1.3162x over previous
import jax
import jax.numpy as jnp
from jax import lax
from jax.experimental import pallas as pl
from jax.experimental.pallas import tpu as pltpu

C = 8


def kernel(x):
    m, n = x.shape
    half = m // 2
    rpc = half // C

    def body(x_ref, out_ref, copy_sem, xsend, xrecv, zsend, zrecv):
        mx = lax.axis_index("x")
        my = lax.axis_index("y")
        mz = lax.axis_index("z")
        xpeer = (1 - mx, my, mz)
        zpartner = (mx, my, 1 - mz)

        barrier_sem = pltpu.get_barrier_semaphore()
        for nbr in (xpeer, zpartner):
            pl.semaphore_signal(
                barrier_sem, inc=1,
                device_id=nbr, device_id_type=pl.DeviceIdType.MESH,
            )
        pl.semaphore_wait(barrier_sem, 2)

        x_rdmas = []
        for c in range(C):
            src_off = mz * half + c * rpc
            dst_off = mx * m + mz * half + c * rpc
            rdma = pltpu.make_async_remote_copy(
                src_ref=x_ref.at[pl.ds(src_off, rpc), :],
                dst_ref=out_ref.at[pl.ds(dst_off, rpc), :],
                send_sem=xsend.at[c],
                recv_sem=xrecv.at[c],
                device_id=xpeer,
                device_id_type=pl.DeviceIdType.MESH,
            )
            rdma.start()
            x_rdmas.append(rdma)

        local = pltpu.make_async_copy(
            x_ref, out_ref.at[pl.ds(mx * m, m), :], copy_sem
        )
        local.start()

        z_rdmas = []
        for c in range(C):
            x_rdmas[c].wait_recv()
            fwd_off = (1 - mx) * m + mz * half + c * rpc
            rdma = pltpu.make_async_remote_copy(
                src_ref=out_ref.at[pl.ds(fwd_off, rpc), :],
                dst_ref=out_ref.at[pl.ds(fwd_off, rpc), :],
                send_sem=zsend.at[c],
                recv_sem=zrecv.at[c],
                device_id=zpartner,
                device_id_type=pl.DeviceIdType.MESH,
            )
            rdma.start()
            z_rdmas.append(rdma)

        for c in range(C):
            z_rdmas[c].wait_recv()
        for c in range(C):
            x_rdmas[c].wait_send()
            z_rdmas[c].wait_send()
        local.wait()

    return pl.pallas_call(
        body,
        out_shape=jax.ShapeDtypeStruct((2 * m, n), x.dtype),
        in_specs=[pl.BlockSpec(memory_space=pltpu.VMEM)],
        out_specs=pl.BlockSpec(memory_space=pltpu.VMEM),
        scratch_shapes=[
            pltpu.SemaphoreType.DMA,
            pltpu.SemaphoreType.DMA((C,)),
            pltpu.SemaphoreType.DMA((C,)),
            pltpu.SemaphoreType.DMA((C,)),
            pltpu.SemaphoreType.DMA((C,)),
        ],
        compiler_params=pltpu.CompilerParams(collective_id=0),
    )(x)


# device time: 22409 ns/iter; 1.3395x vs baseline; 1.0177x over previous
import jax
import jax.numpy as jnp
from jax import lax
from jax.experimental import pallas as pl
from jax.experimental.pallas import tpu as pltpu

CHUNK_ROWS = [32] * 16
assert sum(CHUNK_ROWS) == 512
C = len(CHUNK_ROWS)
CHUNK_OFFS = [sum(CHUNK_ROWS[:i]) for i in range(C)]


def kernel(x):
    m, n = x.shape
    half = m // 2

    def body(x_ref, out_ref, copy_sem, xsend, xrecv, zsend, zrecv):
        mx = lax.axis_index("x")
        my = lax.axis_index("y")
        mz = lax.axis_index("z")
        xpeer = (1 - mx, my, mz)
        zpartner = (mx, my, 1 - mz)

        barrier_sem = pltpu.get_barrier_semaphore()
        for nbr in (xpeer, zpartner):
            pl.semaphore_signal(
                barrier_sem, inc=1,
                device_id=nbr, device_id_type=pl.DeviceIdType.MESH,
            )
        pl.semaphore_wait(barrier_sem, 2)

        x_rdmas = []
        for c in range(C):
            rows = CHUNK_ROWS[c]
            src_off = mz * half + CHUNK_OFFS[c]
            dst_off = mx * m + mz * half + CHUNK_OFFS[c]
            rdma = pltpu.make_async_remote_copy(
                src_ref=x_ref.at[pl.ds(src_off, rows), :],
                dst_ref=out_ref.at[pl.ds(dst_off, rows), :],
                send_sem=xsend.at[c],
                recv_sem=xrecv.at[c],
                device_id=xpeer,
                device_id_type=pl.DeviceIdType.MESH,
            )
            rdma.start()
            x_rdmas.append(rdma)

        local = pltpu.make_async_copy(
            x_ref, out_ref.at[pl.ds(mx * m, m), :], copy_sem
        )
        local.start()

        z_rdmas = []
        for c in range(C):
            x_rdmas[c].wait_recv()
            rows = CHUNK_ROWS[c]
            fwd_off = (1 - mx) * m + mz * half + CHUNK_OFFS[c]
            rdma = pltpu.make_async_remote_copy(
                src_ref=out_ref.at[pl.ds(fwd_off, rows), :],
                dst_ref=out_ref.at[pl.ds(fwd_off, rows), :],
                send_sem=zsend.at[c],
                recv_sem=zrecv.at[c],
                device_id=zpartner,
                device_id_type=pl.DeviceIdType.MESH,
            )
            rdma.start()
            z_rdmas.append(rdma)

        for c in range(C):
            z_rdmas[c].wait_recv()
        for c in range(C):
            x_rdmas[c].wait_send()
            z_rdmas[c].wait_send()
        local.wait()

    return pl.pallas_call(
        body,
        out_shape=jax.ShapeDtypeStruct((2 * m, n), x.dtype),
        in_specs=[pl.BlockSpec(memory_space=pltpu.VMEM)],
        out_specs=pl.BlockSpec(memory_space=pltpu.VMEM),
        scratch_shapes=[
            pltpu.SemaphoreType.DMA,
            pltpu.SemaphoreType.DMA((C,)),
            pltpu.SemaphoreType.DMA((C,)),
            pltpu.SemaphoreType.DMA((C,)),
            pltpu.SemaphoreType.DMA((C,)),
        ],
        compiler_params=pltpu.CompilerParams(collective_id=0),
    )(x)
